# Initial kernel scaffold; baseline (speedup 1.0000x reference)
import jax
import jax.numpy as jnp
from jax.experimental import pallas as pl


def kernel(grid_node_features, mesh_node_features, grid2mesh_edge_features, grid2mesh_edge_indices_src, grid2mesh_edge_indices_dst, W1_e, b1_e, W2_e, b2_e, g_e, be_e, W1_n, b1_n, W2_n, b2_n, g_n, be_n, W1_g, b1_g, W2_g, b2_g, g_g, be_g):
    raise NotImplementedError("write your pallas kernel here")


# speedup vs baseline: 3.0525x; 3.0525x over previous
import functools

import jax
import jax.numpy as jnp
from jax import lax
from jax.experimental import pallas as pl
from jax.experimental.pallas import tpu as pltpu
from jax.experimental.pallas import tpu_sc as plsc

N_GRID = 10000
N_MESH = 10000
E = 320000
D = 128
L = 16

NC = 2
NS = 16
NW = NC * NS
EPW = E // NW
CHUNK = 80
NCHUNK = EPW // CHUNK
ZR = 624
ZTAIL = N_MESH - NS * ZR
ZTAIL_OFF = NS * ZR

NODE_BLK = 1000
EDGE_BLK = 2000

def _sc_mesh():
    return plsc.VectorSubcoreMesh(
        core_axis_name="c", subcore_axis_name="s",
        num_cores=NC, num_subcores=NS)


def _layernorm(o, g, b):
    mu = jnp.mean(o, axis=-1, keepdims=True)
    d = o - mu
    var = jnp.mean(d * d, axis=-1, keepdims=True)
    return g * (d * lax.rsqrt(var + 1e-5)) + b


def _silu(x):
    return x * lax.logistic(x)


def _node_pre_body(gn, mn, wb, wc, g1, m1):
    g1[...] = jnp.dot(gn[...], wb[...], preferred_element_type=jnp.float32)
    m1[...] = jnp.dot(mn[...], wc[...], preferred_element_type=jnp.float32)


def _edge_mlp_body(e_ref, gs_ref, wa, w2, b1, b2, g, be, out_ref):
    e = e_ref[...]
    h = _silu(jnp.dot(e, wa[...], preferred_element_type=jnp.float32)
              + gs_ref[...] + b1[...])
    o = jnp.dot(h, w2[...], preferred_element_type=jnp.float32) + b2[...]
    out_ref[...] = _layernorm(o, g[...], be[...]) + e


def _node_post_body(m_ref, p_ref, x_ref, w1na, w1nb, w2n, b1n, b2n, gn, ben,
                    w1g, w2g, b1g, b2g, gg, beg, mout_ref, gout_ref):
    m = m_ref[...]
    agg = p_ref[0] + p_ref[1]
    h = _silu(jnp.dot(m, w1na[...], preferred_element_type=jnp.float32)
              + jnp.dot(agg, w1nb[...], preferred_element_type=jnp.float32)
              + b1n[...])
    o = jnp.dot(h, w2n[...], preferred_element_type=jnp.float32) + b2n[...]
    mout_ref[...] = m + _layernorm(o, gn[...], ben[...])
    x = x_ref[...]
    h2 = _silu(jnp.dot(x, w1g[...], preferred_element_type=jnp.float32)
               + b1g[...])
    o2 = jnp.dot(h2, w2g[...], preferred_element_type=jnp.float32) + b2g[...]
    gout_ref[...] = x + _layernorm(o2, gg[...], beg[...])


def _row_spec(blk):
    return pl.BlockSpec((blk, D), lambda i: (i, 0))


def _full_spec(shape):
    return pl.BlockSpec(shape, lambda i: tuple(0 for _ in shape))


def _sc_gather_body(g1_hbm, m1_hbm, src_hbm, dst_hbm, out_hbm,
                    sidx, didx, gbuf, mbuf, sem1, sem2):
    wid = lax.axis_index("s") * NC + lax.axis_index("c")
    base = wid * EPW

    def step(i, carry):
        b = base + i * CHUNK
        pltpu.sync_copy(src_hbm.at[pl.ds(b, CHUNK)], sidx)
        pltpu.sync_copy(dst_hbm.at[pl.ds(b, CHUNK)], didx)
        cp1 = pltpu.async_copy(g1_hbm.at[sidx], gbuf, sem1)
        cp2 = pltpu.async_copy(m1_hbm.at[didx], mbuf, sem2)
        cp1.wait()
        cp2.wait()

        def add_row(r, c2):
            for j in range(D // L):
                sl = pl.ds(j * L, L)
                gbuf[r, sl] = gbuf[r, sl] + mbuf[r, sl]
            return c2

        lax.fori_loop(0, CHUNK, add_row, 0)
        pltpu.sync_copy(gbuf, out_hbm.at[pl.ds(b, CHUNK)])
        return carry

    lax.fori_loop(0, NCHUNK, step, 0)


def _sc_scatter_body(eo_hbm, dst_hbm, zero_hbm, out_hbm, didx, ebuf, agg):
    c = lax.axis_index("c")
    s = lax.axis_index("s")
    wid = s * NC + c
    base = wid * EPW
    pltpu.sync_copy(zero_hbm.at[pl.ds(s * ZR, ZR)], agg.at[pl.ds(s * ZR, ZR)])

    @pl.when(s == 0)
    def _():
        pltpu.sync_copy(zero_hbm.at[pl.ds(ZTAIL_OFF, ZTAIL)],
                        agg.at[pl.ds(ZTAIL_OFF, ZTAIL)])

    plsc.subcore_barrier()

    def step(i, carry):
        b = base + i * CHUNK
        pltpu.sync_copy(dst_hbm.at[pl.ds(b, CHUNK)], didx)
        pltpu.sync_copy(eo_hbm.at[pl.ds(b, CHUNK)], ebuf)
        pltpu.sync_copy(ebuf, agg.at[didx], add=True)
        return carry

    lax.fori_loop(0, NCHUNK, step, 0)
    plsc.subcore_barrier()
    pltpu.sync_copy(agg.at[pl.ds(s * ZR, ZR)], out_hbm.at[c, pl.ds(s * ZR, ZR)])

    @pl.when(s == 0)
    def _():
        pltpu.sync_copy(agg.at[pl.ds(ZTAIL_OFF, ZTAIL)],
                        out_hbm.at[c, pl.ds(ZTAIL_OFF, ZTAIL)])


def _sc_gather(g1, m1, src, dst):
    return pl.kernel(
        _sc_gather_body,
        out_type=jax.ShapeDtypeStruct((E, D), jnp.float32),
        mesh=_sc_mesh(),
        scratch_types=[
            pltpu.VMEM((CHUNK,), jnp.int32),
            pltpu.VMEM((CHUNK,), jnp.int32),
            pltpu.VMEM((CHUNK, D), jnp.float32),
            pltpu.VMEM((CHUNK, D), jnp.float32),
            pltpu.SemaphoreType.DMA,
            pltpu.SemaphoreType.DMA,
        ],
    )(g1, m1, src, dst)


def _sc_scatter(e_out, dst, zeros):
    return pl.kernel(
        _sc_scatter_body,
        out_type=jax.ShapeDtypeStruct((NC, N_MESH, D), jnp.float32),
        mesh=_sc_mesh(),
        scratch_types=[
            pltpu.VMEM((CHUNK,), jnp.int32),
            pltpu.VMEM((CHUNK, D), jnp.float32),
            pltpu.VMEM_SHARED((N_MESH, D), jnp.float32),
        ],
    )(e_out, dst, zeros)


def kernel(grid_node_features, mesh_node_features, grid2mesh_edge_features,
           grid2mesh_edge_indices_src, grid2mesh_edge_indices_dst,
           W1_e, b1_e, W2_e, b2_e, g_e, be_e,
           W1_n, b1_n, W2_n, b2_n, g_n, be_n,
           W1_g, b1_g, W2_g, b2_g, g_g, be_g):
    f32 = jnp.float32
    row = lambda v: v.reshape(1, D)
    W1a, W1b, W1c = W1_e[:D], W1_e[D:2 * D], W1_e[2 * D:]
    W1na, W1nb = W1_n[:D], W1_n[D:]

    g1_m1 = pl.pallas_call(
        _node_pre_body,
        grid=(N_GRID // NODE_BLK,),
        in_specs=[_row_spec(NODE_BLK), _row_spec(NODE_BLK),
                  _full_spec((D, D)), _full_spec((D, D))],
        out_specs=[_row_spec(NODE_BLK), _row_spec(NODE_BLK)],
        out_shape=[jax.ShapeDtypeStruct((N_GRID, D), f32),
                   jax.ShapeDtypeStruct((N_MESH, D), f32)],
    )(grid_node_features, mesh_node_features, W1b, W1c)
    g1, m1 = g1_m1

    gsum = _sc_gather(g1, m1, grid2mesh_edge_indices_src,
                      grid2mesh_edge_indices_dst)

    e_out = pl.pallas_call(
        _edge_mlp_body,
        grid=(E // EDGE_BLK,),
        in_specs=[_row_spec(EDGE_BLK), _row_spec(EDGE_BLK),
                  _full_spec((D, D)), _full_spec((D, D)),
                  _full_spec((1, D)), _full_spec((1, D)),
                  _full_spec((1, D)), _full_spec((1, D))],
        out_specs=_row_spec(EDGE_BLK),
        out_shape=jax.ShapeDtypeStruct((E, D), f32),
    )(grid2mesh_edge_features, gsum, W1a, W2_e,
      row(b1_e), row(b2_e), row(g_e), row(be_e))

    partials = _sc_scatter(e_out, grid2mesh_edge_indices_dst,
                           jnp.zeros((N_MESH, D), f32))

    mesh_out, grid_out = pl.pallas_call(
        _node_post_body,
        grid=(N_MESH // NODE_BLK,),
        in_specs=[_row_spec(NODE_BLK),
                  pl.BlockSpec((NC, NODE_BLK, D), lambda i: (0, i, 0)),
                  _row_spec(NODE_BLK),
                  _full_spec((D, D)), _full_spec((D, D)), _full_spec((D, D)),
                  _full_spec((1, D)), _full_spec((1, D)),
                  _full_spec((1, D)), _full_spec((1, D)),
                  _full_spec((D, D)), _full_spec((D, D)),
                  _full_spec((1, D)), _full_spec((1, D)),
                  _full_spec((1, D)), _full_spec((1, D))],
        out_specs=[_row_spec(NODE_BLK), _row_spec(NODE_BLK)],
        out_shape=[jax.ShapeDtypeStruct((N_MESH, D), f32),
                   jax.ShapeDtypeStruct((N_GRID, D), f32)],
    )(mesh_node_features, partials, grid_node_features,
      W1na, W1nb, W2_n, row(b1_n), row(b2_n), row(g_n), row(be_n),
      W1_g, W2_g, row(b1_g), row(b2_g), row(g_g), row(be_g))

    return (grid_out, mesh_out)


# speedup vs baseline: 3.2604x; 1.0681x over previous
import functools

import jax
import jax.numpy as jnp
from jax import lax
from jax.experimental import pallas as pl
from jax.experimental.pallas import tpu as pltpu
from jax.experimental.pallas import tpu_sc as plsc

N_GRID = 10000
N_MESH = 10000
E = 320000
D = 128
L = 16

NC = 2
NS = 16
NW = NC * NS
EPW = E // NW
CHUNK = 80
NCHUNK = EPW // CHUNK
ZR = 624
ZTAIL = N_MESH - NS * ZR
ZTAIL_OFF = NS * ZR

NODE_BLK = 1000
EDGE_BLK = 2000

def _sc_mesh():
    return plsc.VectorSubcoreMesh(
        core_axis_name="c", subcore_axis_name="s",
        num_cores=NC, num_subcores=NS)


def _layernorm(o, g, b):
    mu = jnp.mean(o, axis=-1, keepdims=True)
    d = o - mu
    var = jnp.mean(d * d, axis=-1, keepdims=True)
    return g * (d * lax.rsqrt(var + 1e-5)) + b


def _silu(x):
    return x * lax.logistic(x)


def _node_pre_body(gn, mn, wb, wc, g1, m1):
    g1[...] = jnp.dot(gn[...], wb[...], preferred_element_type=jnp.float32)
    m1[...] = jnp.dot(mn[...], wc[...], preferred_element_type=jnp.float32)


def _edge_mlp_body(e_ref, gs_ref, wa, w2, b1, b2, g, be, out_ref):
    e = e_ref[...]
    h = _silu(jnp.dot(e, wa[...], preferred_element_type=jnp.float32)
              + gs_ref[...] + b1[...])
    o = jnp.dot(h, w2[...], preferred_element_type=jnp.float32) + b2[...]
    out_ref[...] = _layernorm(o, g[...], be[...]) + e


def _node_post_body(m_ref, p_ref, x_ref, w1na, w1nb, w2n, b1n, b2n, gn, ben,
                    w1g, w2g, b1g, b2g, gg, beg, mout_ref, gout_ref):
    m = m_ref[...]
    agg = p_ref[0] + p_ref[1]
    h = _silu(jnp.dot(m, w1na[...], preferred_element_type=jnp.float32)
              + jnp.dot(agg, w1nb[...], preferred_element_type=jnp.float32)
              + b1n[...])
    o = jnp.dot(h, w2n[...], preferred_element_type=jnp.float32) + b2n[...]
    mout_ref[...] = m + _layernorm(o, gn[...], ben[...])
    x = x_ref[...]
    h2 = _silu(jnp.dot(x, w1g[...], preferred_element_type=jnp.float32)
               + b1g[...])
    o2 = jnp.dot(h2, w2g[...], preferred_element_type=jnp.float32) + b2g[...]
    gout_ref[...] = x + _layernorm(o2, gg[...], beg[...])


def _row_spec(blk):
    return pl.BlockSpec((blk, D), lambda i: (i, 0))


def _full_spec(shape):
    return pl.BlockSpec(shape, lambda i: tuple(0 for _ in shape))


def _sc_gather_body(g1_hbm, m1_hbm, src_hbm, dst_hbm, out_hbm,
                    sidx, didx, gbuf, mbuf, sem1, sem2):
    wid = lax.axis_index("s") * NC + lax.axis_index("c")
    base = wid * EPW
    pltpu.sync_copy(src_hbm.at[pl.ds(base, EPW)], sidx)
    pltpu.sync_copy(dst_hbm.at[pl.ds(base, EPW)], didx)

    def fire(i, p):
        isl = pl.ds(i * CHUNK, CHUNK)
        pltpu.async_copy(g1_hbm.at[sidx.at[isl]], gbuf.at[p], sem1)
        pltpu.async_copy(m1_hbm.at[didx.at[isl]], mbuf.at[p], sem2)

    def drain(p):
        pltpu.make_async_copy(g1_hbm.at[sidx.at[pl.ds(0, CHUNK)]],
                              gbuf.at[p], sem1).wait()
        pltpu.make_async_copy(m1_hbm.at[didx.at[pl.ds(0, CHUNK)]],
                              mbuf.at[p], sem2).wait()

    def process(i, p):
        def add_row(r, c2):
            for j in range(D // L):
                sl = pl.ds(j * L, L)
                gbuf[p, r, sl] = gbuf[p, r, sl] + mbuf[p, r, sl]
            return c2

        lax.fori_loop(0, CHUNK, add_row, 0)
        pltpu.sync_copy(gbuf.at[p], out_hbm.at[pl.ds(base + i * CHUNK, CHUNK)])

    fire(0, 0)

    def step(i, carry):
        p = lax.rem(i, 2)

        @pl.when(i + 1 < NCHUNK)
        def _():
            fire(i + 1, 1 - p)

        drain(p)
        process(i, p)
        return carry

    lax.fori_loop(0, NCHUNK, step, 0)


def _sc_scatter_body(eo_hbm, dst_hbm, zero_hbm, out_hbm, didx, ebuf, agg,
                     semi, seme):
    c = lax.axis_index("c")
    s = lax.axis_index("s")
    wid = s * NC + c
    base = wid * EPW
    pltpu.sync_copy(zero_hbm.at[pl.ds(s * ZR, ZR)], agg.at[pl.ds(s * ZR, ZR)])

    @pl.when(s == 0)
    def _():
        pltpu.sync_copy(zero_hbm.at[pl.ds(ZTAIL_OFF, ZTAIL)],
                        agg.at[pl.ds(ZTAIL_OFF, ZTAIL)])

    plsc.subcore_barrier()

    def fire(i, p):
        b = base + i * CHUNK
        pltpu.async_copy(dst_hbm.at[pl.ds(b, CHUNK)], didx.at[p], semi)
        pltpu.async_copy(eo_hbm.at[pl.ds(b, CHUNK)], ebuf.at[p], seme)

    def drain(p):
        pltpu.make_async_copy(dst_hbm.at[pl.ds(0, CHUNK)],
                              didx.at[p], semi).wait()
        pltpu.make_async_copy(eo_hbm.at[pl.ds(0, CHUNK)],
                              ebuf.at[p], seme).wait()

    fire(0, 0)

    def step(i, carry):
        p = lax.rem(i, 2)

        @pl.when(i + 1 < NCHUNK)
        def _():
            fire(i + 1, 1 - p)

        drain(p)
        pltpu.sync_copy(ebuf.at[p], agg.at[didx.at[p]], add=True)
        return carry

    lax.fori_loop(0, NCHUNK, step, 0)
    plsc.subcore_barrier()
    pltpu.sync_copy(agg.at[pl.ds(s * ZR, ZR)], out_hbm.at[c, pl.ds(s * ZR, ZR)])

    @pl.when(s == 0)
    def _():
        pltpu.sync_copy(agg.at[pl.ds(ZTAIL_OFF, ZTAIL)],
                        out_hbm.at[c, pl.ds(ZTAIL_OFF, ZTAIL)])


def _sc_gather(g1, m1, src, dst):
    return pl.kernel(
        _sc_gather_body,
        out_type=jax.ShapeDtypeStruct((E, D), jnp.float32),
        mesh=_sc_mesh(),
        scratch_types=[
            pltpu.VMEM((EPW,), jnp.int32),
            pltpu.VMEM((EPW,), jnp.int32),
            pltpu.VMEM((2, CHUNK, D), jnp.float32),
            pltpu.VMEM((2, CHUNK, D), jnp.float32),
            pltpu.SemaphoreType.DMA,
            pltpu.SemaphoreType.DMA,
        ],
    )(g1, m1, src, dst)


def _sc_scatter(e_out, dst, zeros):
    return pl.kernel(
        _sc_scatter_body,
        out_type=jax.ShapeDtypeStruct((NC, N_MESH, D), jnp.float32),
        mesh=_sc_mesh(),
        scratch_types=[
            pltpu.VMEM((2, CHUNK), jnp.int32),
            pltpu.VMEM((2, CHUNK, D), jnp.float32),
            pltpu.VMEM_SHARED((N_MESH, D), jnp.float32),
            pltpu.SemaphoreType.DMA,
            pltpu.SemaphoreType.DMA,
        ],
    )(e_out, dst, zeros)


def kernel(grid_node_features, mesh_node_features, grid2mesh_edge_features,
           grid2mesh_edge_indices_src, grid2mesh_edge_indices_dst,
           W1_e, b1_e, W2_e, b2_e, g_e, be_e,
           W1_n, b1_n, W2_n, b2_n, g_n, be_n,
           W1_g, b1_g, W2_g, b2_g, g_g, be_g):
    f32 = jnp.float32
    row = lambda v: v.reshape(1, D)
    W1a, W1b, W1c = W1_e[:D], W1_e[D:2 * D], W1_e[2 * D:]
    W1na, W1nb = W1_n[:D], W1_n[D:]

    g1_m1 = pl.pallas_call(
        _node_pre_body,
        grid=(N_GRID // NODE_BLK,),
        in_specs=[_row_spec(NODE_BLK), _row_spec(NODE_BLK),
                  _full_spec((D, D)), _full_spec((D, D))],
        out_specs=[_row_spec(NODE_BLK), _row_spec(NODE_BLK)],
        out_shape=[jax.ShapeDtypeStruct((N_GRID, D), f32),
                   jax.ShapeDtypeStruct((N_MESH, D), f32)],
    )(grid_node_features, mesh_node_features, W1b, W1c)
    g1, m1 = g1_m1

    gsum = _sc_gather(g1, m1, grid2mesh_edge_indices_src,
                      grid2mesh_edge_indices_dst)

    e_out = pl.pallas_call(
        _edge_mlp_body,
        grid=(E // EDGE_BLK,),
        in_specs=[_row_spec(EDGE_BLK), _row_spec(EDGE_BLK),
                  _full_spec((D, D)), _full_spec((D, D)),
                  _full_spec((1, D)), _full_spec((1, D)),
                  _full_spec((1, D)), _full_spec((1, D))],
        out_specs=_row_spec(EDGE_BLK),
        out_shape=jax.ShapeDtypeStruct((E, D), f32),
    )(grid2mesh_edge_features, gsum, W1a, W2_e,
      row(b1_e), row(b2_e), row(g_e), row(be_e))

    partials = _sc_scatter(e_out, grid2mesh_edge_indices_dst,
                           jnp.zeros((N_MESH, D), f32))

    mesh_out, grid_out = pl.pallas_call(
        _node_post_body,
        grid=(N_MESH // NODE_BLK,),
        in_specs=[_row_spec(NODE_BLK),
                  pl.BlockSpec((NC, NODE_BLK, D), lambda i: (0, i, 0)),
                  _row_spec(NODE_BLK),
                  _full_spec((D, D)), _full_spec((D, D)), _full_spec((D, D)),
                  _full_spec((1, D)), _full_spec((1, D)),
                  _full_spec((1, D)), _full_spec((1, D)),
                  _full_spec((D, D)), _full_spec((D, D)),
                  _full_spec((1, D)), _full_spec((1, D)),
                  _full_spec((1, D)), _full_spec((1, D))],
        out_specs=[_row_spec(NODE_BLK), _row_spec(NODE_BLK)],
        out_shape=[jax.ShapeDtypeStruct((N_MESH, D), f32),
                   jax.ShapeDtypeStruct((N_GRID, D), f32)],
    )(mesh_node_features, partials, grid_node_features,
      W1na, W1nb, W2_n, row(b1_n), row(b2_n), row(g_n), row(be_n),
      W1_g, W2_g, row(b1_g), row(b2_g), row(g_g), row(be_g))

    return (grid_out, mesh_out)


# speedup vs baseline: 5.0520x; 1.5495x over previous
import functools

import jax
import jax.numpy as jnp
from jax import lax
from jax.experimental import pallas as pl
from jax.experimental.pallas import tpu as pltpu
from jax.experimental.pallas import tpu_sc as plsc

N_GRID = 10000
N_MESH = 10000
E = 320000
D = 128
L = 16

NC = 2
NS = 16
NW = NC * NS
EPW = E // NW
CHUNK = 80
NCHUNK = EPW // CHUNK
ZR = 624
ZTAIL = N_MESH - NS * ZR
ZTAIL_OFF = NS * ZR

NODE_BLK = 1000
EDGE_BLK = 2000

def _sc_mesh():
    return plsc.VectorSubcoreMesh(
        core_axis_name="c", subcore_axis_name="s",
        num_cores=NC, num_subcores=NS)


def _layernorm(o, g, b):
    mu = jnp.mean(o, axis=-1, keepdims=True)
    d = o - mu
    var = jnp.mean(d * d, axis=-1, keepdims=True)
    return g * (d * lax.rsqrt(var + 1e-5)) + b


def _silu(x):
    return x * lax.logistic(x)


def _node_pre_body(gn, mn, wb, wc, g1, m1):
    g1[...] = jnp.dot(gn[...], wb[...], preferred_element_type=jnp.float32)
    m1[...] = jnp.dot(mn[...], wc[...], preferred_element_type=jnp.float32)


def _edge_mlp_body(e_ref, gs_ref, wa, w2, b1, b2, g, be, out_ref):
    e = e_ref[...]
    h = _silu(jnp.dot(e, wa[...], preferred_element_type=jnp.float32)
              + gs_ref[...] + b1[...])
    o = jnp.dot(h, w2[...], preferred_element_type=jnp.float32) + b2[...]
    out_ref[...] = _layernorm(o, g[...], be[...]) + e


def _node_post_body(m_ref, p_ref, x_ref, w1na, w1nb, w2n, b1n, b2n, gn, ben,
                    w1g, w2g, b1g, b2g, gg, beg, mout_ref, gout_ref):
    m = m_ref[...]
    agg = p_ref[0] + p_ref[1]
    h = _silu(jnp.dot(m, w1na[...], preferred_element_type=jnp.float32)
              + jnp.dot(agg, w1nb[...], preferred_element_type=jnp.float32)
              + b1n[...])
    o = jnp.dot(h, w2n[...], preferred_element_type=jnp.float32) + b2n[...]
    mout_ref[...] = m + _layernorm(o, gn[...], ben[...])
    x = x_ref[...]
    h2 = _silu(jnp.dot(x, w1g[...], preferred_element_type=jnp.float32)
               + b1g[...])
    o2 = jnp.dot(h2, w2g[...], preferred_element_type=jnp.float32) + b2g[...]
    gout_ref[...] = x + _layernorm(o2, gg[...], beg[...])


def _row_spec(blk):
    return pl.BlockSpec((blk, D), lambda i: (i, 0))


def _full_spec(shape):
    return pl.BlockSpec(shape, lambda i: tuple(0 for _ in shape))


def _sc_gather_body(g1_hbm, m1_hbm, src_hbm, dst_hbm, out_hbm,
                    sidx, didx, gbuf, mbuf, sem1, sem2):
    wid = lax.axis_index("s") * NC + lax.axis_index("c")
    base = wid * EPW
    pltpu.sync_copy(src_hbm.at[pl.ds(base, EPW)], sidx)
    pltpu.sync_copy(dst_hbm.at[pl.ds(base, EPW)], didx)

    def fire(i, p):
        isl = pl.ds(i * CHUNK, CHUNK)
        pltpu.async_copy(g1_hbm.at[sidx.at[isl]], gbuf.at[p], sem1)
        pltpu.async_copy(m1_hbm.at[didx.at[isl]], mbuf.at[p], sem2)

    def drain(p):
        pltpu.make_async_copy(g1_hbm.at[sidx.at[pl.ds(0, CHUNK)]],
                              gbuf.at[p], sem1).wait()
        pltpu.make_async_copy(m1_hbm.at[didx.at[pl.ds(0, CHUNK)]],
                              mbuf.at[p], sem2).wait()

    def process(i, p):
        def add_row(r, c2):
            for j in range(D // L):
                sl = pl.ds(j * L, L)
                gbuf[p, r, sl] = gbuf[p, r, sl] + mbuf[p, r, sl]
            return c2

        lax.fori_loop(0, CHUNK, add_row, 0)
        pltpu.sync_copy(gbuf.at[p], out_hbm.at[pl.ds(base + i * CHUNK, CHUNK)])

    fire(0, 0)

    def pair(j, carry):
        i0 = 2 * j
        fire(i0 + 1, 1)
        drain(0)
        process(i0, 0)

        @pl.when(i0 + 2 < NCHUNK)
        def _():
            fire(i0 + 2, 0)

        drain(1)
        process(i0 + 1, 1)
        return carry

    lax.fori_loop(0, NCHUNK // 2, pair, 0)
    if NCHUNK % 2 == 1:
        drain(0)
        process(NCHUNK - 1, 0)


def _sc_scatter_body(eo_hbm, dst_hbm, zero_hbm, out_hbm, didx, ebuf, agg,
                     semi, seme):
    c = lax.axis_index("c")
    s = lax.axis_index("s")
    wid = s * NC + c
    base = wid * EPW
    pltpu.sync_copy(zero_hbm.at[pl.ds(s * ZR, ZR)], agg.at[pl.ds(s * ZR, ZR)])

    @pl.when(s == 0)
    def _():
        pltpu.sync_copy(zero_hbm.at[pl.ds(ZTAIL_OFF, ZTAIL)],
                        agg.at[pl.ds(ZTAIL_OFF, ZTAIL)])

    plsc.subcore_barrier()

    def fire(i, p):
        b = base + i * CHUNK
        pltpu.async_copy(dst_hbm.at[pl.ds(b, CHUNK)], didx.at[p], semi)
        pltpu.async_copy(eo_hbm.at[pl.ds(b, CHUNK)], ebuf.at[p], seme)

    def drain(p):
        pltpu.make_async_copy(dst_hbm.at[pl.ds(0, CHUNK)],
                              didx.at[p], semi).wait()
        pltpu.make_async_copy(eo_hbm.at[pl.ds(0, CHUNK)],
                              ebuf.at[p], seme).wait()

    fire(0, 0)

    def step(i, carry):
        p = lax.rem(i, 2)

        @pl.when(i + 1 < NCHUNK)
        def _():
            fire(i + 1, 1 - p)

        drain(p)
        pltpu.sync_copy(ebuf.at[p], agg.at[didx.at[p]], add=True)
        return carry

    lax.fori_loop(0, NCHUNK, step, 0)
    plsc.subcore_barrier()
    pltpu.sync_copy(agg.at[pl.ds(s * ZR, ZR)], out_hbm.at[c, pl.ds(s * ZR, ZR)])

    @pl.when(s == 0)
    def _():
        pltpu.sync_copy(agg.at[pl.ds(ZTAIL_OFF, ZTAIL)],
                        out_hbm.at[c, pl.ds(ZTAIL_OFF, ZTAIL)])


def _sc_gather(g1, m1, src, dst):
    return pl.kernel(
        _sc_gather_body,
        out_type=jax.ShapeDtypeStruct((E, D), jnp.float32),
        mesh=_sc_mesh(),
        scratch_types=[
            pltpu.VMEM((EPW,), jnp.int32),
            pltpu.VMEM((EPW,), jnp.int32),
            pltpu.VMEM((2, CHUNK, D), jnp.float32),
            pltpu.VMEM((2, CHUNK, D), jnp.float32),
            pltpu.SemaphoreType.DMA,
            pltpu.SemaphoreType.DMA,
        ],
    )(g1, m1, src, dst)


def _sc_scatter(e_out, dst, zeros):
    return pl.kernel(
        _sc_scatter_body,
        out_type=jax.ShapeDtypeStruct((NC, N_MESH, D), jnp.float32),
        mesh=_sc_mesh(),
        scratch_types=[
            pltpu.VMEM((2, CHUNK), jnp.int32),
            pltpu.VMEM((2, CHUNK, D), jnp.float32),
            pltpu.VMEM_SHARED((N_MESH, D), jnp.float32),
            pltpu.SemaphoreType.DMA,
            pltpu.SemaphoreType.DMA,
        ],
    )(e_out, dst, zeros)


def kernel(grid_node_features, mesh_node_features, grid2mesh_edge_features,
           grid2mesh_edge_indices_src, grid2mesh_edge_indices_dst,
           W1_e, b1_e, W2_e, b2_e, g_e, be_e,
           W1_n, b1_n, W2_n, b2_n, g_n, be_n,
           W1_g, b1_g, W2_g, b2_g, g_g, be_g):
    f32 = jnp.float32
    row = lambda v: v.reshape(1, D)
    W1a, W1b, W1c = W1_e[:D], W1_e[D:2 * D], W1_e[2 * D:]
    W1na, W1nb = W1_n[:D], W1_n[D:]

    g1_m1 = pl.pallas_call(
        _node_pre_body,
        grid=(N_GRID // NODE_BLK,),
        in_specs=[_row_spec(NODE_BLK), _row_spec(NODE_BLK),
                  _full_spec((D, D)), _full_spec((D, D))],
        out_specs=[_row_spec(NODE_BLK), _row_spec(NODE_BLK)],
        out_shape=[jax.ShapeDtypeStruct((N_GRID, D), f32),
                   jax.ShapeDtypeStruct((N_MESH, D), f32)],
    )(grid_node_features, mesh_node_features, W1b, W1c)
    g1, m1 = g1_m1

    gsum = _sc_gather(g1, m1, grid2mesh_edge_indices_src,
                      grid2mesh_edge_indices_dst)

    e_out = pl.pallas_call(
        _edge_mlp_body,
        grid=(E // EDGE_BLK,),
        in_specs=[_row_spec(EDGE_BLK), _row_spec(EDGE_BLK),
                  _full_spec((D, D)), _full_spec((D, D)),
                  _full_spec((1, D)), _full_spec((1, D)),
                  _full_spec((1, D)), _full_spec((1, D))],
        out_specs=_row_spec(EDGE_BLK),
        out_shape=jax.ShapeDtypeStruct((E, D), f32),
    )(grid2mesh_edge_features, gsum, W1a, W2_e,
      row(b1_e), row(b2_e), row(g_e), row(be_e))

    partials = _sc_scatter(e_out, grid2mesh_edge_indices_dst,
                           jnp.zeros((N_MESH, D), f32))

    mesh_out, grid_out = pl.pallas_call(
        _node_post_body,
        grid=(N_MESH // NODE_BLK,),
        in_specs=[_row_spec(NODE_BLK),
                  pl.BlockSpec((NC, NODE_BLK, D), lambda i: (0, i, 0)),
                  _row_spec(NODE_BLK),
                  _full_spec((D, D)), _full_spec((D, D)), _full_spec((D, D)),
                  _full_spec((1, D)), _full_spec((1, D)),
                  _full_spec((1, D)), _full_spec((1, D)),
                  _full_spec((D, D)), _full_spec((D, D)),
                  _full_spec((1, D)), _full_spec((1, D)),
                  _full_spec((1, D)), _full_spec((1, D))],
        out_specs=[_row_spec(NODE_BLK), _row_spec(NODE_BLK)],
        out_shape=[jax.ShapeDtypeStruct((N_MESH, D), f32),
                   jax.ShapeDtypeStruct((N_GRID, D), f32)],
    )(mesh_node_features, partials, grid_node_features,
      W1na, W1nb, W2_n, row(b1_n), row(b2_n), row(g_n), row(be_n),
      W1_g, W2_g, row(b1_g), row(b2_g), row(g_g), row(be_g))

    return (grid_out, mesh_out)
